# baseline (speedup 1.0000x reference)
import jax
import jax.numpy as jnp
from jax import lax
from jax.experimental import pallas as pl
from jax.experimental.pallas import tpu as pltpu, tpu_sc as plsc

RHO = 70.0
T, N = 1025, 20000
NPOS = 65536
SPAIRS = 100
TM1 = T - 1
NW = 32
PP = NPOS // NW
RT = TM1 // NW
SS = RT * SPAIRS
L = 16


def _sigmoid(z):
    return 1.0 / (1.0 + jnp.exp(-z))


def _body(x_hbm, th_hbm, tp_hbm, up_hbm, vp_hbm, us_hbm, vs_hbm,
          kpos_hbm, kneg_hbm,
          th_v, tp_v, up_v, vp_v, iu_v, iv_v, gu_v, gv_v,
          su_v, sv_v, siu_v, siv_v, sgu_v, sgv_v, op_v, on_v, sem):
    wid = lax.axis_index("s") * 2 + lax.axis_index("c")

    pltpu.sync_copy(th_hbm, th_v)
    eps = _sigmoid(th_v[...]) * 0.5

    base = wid * PP
    pltpu.sync_copy(tp_hbm.at[pl.ds(base, PP)], tp_v)
    pltpu.sync_copy(up_hbm.at[pl.ds(base, PP)], up_v)
    pltpu.sync_copy(vp_hbm.at[pl.ds(base, PP)], vp_v)

    def pos_idx(k, c):
        sl = pl.ds(k * L, L)
        roff = tp_v[sl] * N
        iu_v[sl] = roff + up_v[sl]
        iv_v[sl] = roff + vp_v[sl]
        return c
    lax.fori_loop(0, PP // L, pos_idx, 0)

    cu = pltpu.async_copy(x_hbm.at[iu_v], gu_v, sem)
    cv = pltpu.async_copy(x_hbm.at[iv_v], gv_v, sem)
    cu.wait()
    cv.wait()

    def pos_kap(k, c):
        sl = pl.ds(k * L, L)
        d = gu_v[sl] - gv_v[sl]
        op_v[sl] = _sigmoid(RHO * (eps - jnp.abs(d)))
        return c
    lax.fori_loop(0, PP // L, pos_kap, 0)
    pltpu.sync_copy(op_v, kpos_hbm.at[pl.ds(base, PP)])

    pltpu.sync_copy(us_hbm.at[wid], su_v)
    pltpu.sync_copy(vs_hbm.at[wid], sv_v)

    iota = lax.iota(jnp.int32, L)
    t0 = (wid * RT + iota) * N
    t1 = (wid * RT + L + iota) * N

    def samp_idx(j, c):
        b = j * 2 * L
        s0 = pl.ds(b, L)
        s1 = pl.ds(b + L, L)
        siu_v[s0] = su_v[s0] + t0
        siu_v[s1] = su_v[s1] + t1
        siv_v[s0] = sv_v[s0] + t0
        siv_v[s1] = sv_v[s1] + t1
        return c
    lax.fori_loop(0, SPAIRS, samp_idx, 0)

    gsu = pltpu.async_copy(x_hbm.at[siu_v], sgu_v, sem)
    gsv = pltpu.async_copy(x_hbm.at[siv_v], sgv_v, sem)
    gsu.wait()
    gsv.wait()

    def samp_kap(j, acc):
        a0, a1 = acc
        b = j * 2 * L
        s0 = pl.ds(b, L)
        s1 = pl.ds(b + L, L)
        d0 = sgu_v[s0] - sgv_v[s0]
        d1 = sgu_v[s1] - sgv_v[s1]
        a0 = a0 + _sigmoid(RHO * (eps - jnp.abs(d0)))
        a1 = a1 + _sigmoid(RHO * (eps - jnp.abs(d1)))
        return (a0, a1)
    zero = jnp.zeros((L,), jnp.float32)
    a0, a1 = lax.fori_loop(0, SPAIRS, samp_kap, (zero, zero))

    on_v[pl.ds(0, L)] = 1.0 - a0 * (1.0 / SPAIRS)
    on_v[pl.ds(L, L)] = 1.0 - a1 * (1.0 / SPAIRS)
    pltpu.sync_copy(on_v, kneg_hbm.at[pl.ds(wid * RT, RT)])


def kernel(X, theta, u_pos, v_pos, t_pos, u_sample, v_sample):
    x_flat = X.reshape(-1)
    th16 = jnp.broadcast_to(theta.astype(jnp.float32), (L,))
    us_p = u_sample.reshape(NW, RT, SPAIRS).transpose(0, 2, 1).reshape(NW, SS)
    vs_p = v_sample.reshape(NW, RT, SPAIRS).transpose(0, 2, 1).reshape(NW, SS)

    mesh = plsc.VectorSubcoreMesh(core_axis_name="c", subcore_axis_name="s")
    run = pl.kernel(
        _body,
        out_type=(
            jax.ShapeDtypeStruct((NPOS,), jnp.float32),
            jax.ShapeDtypeStruct((TM1,), jnp.float32),
        ),
        mesh=mesh,
        scratch_types=[
            pltpu.VMEM((L,), jnp.float32),
            pltpu.VMEM((PP,), jnp.int32),
            pltpu.VMEM((PP,), jnp.int32),
            pltpu.VMEM((PP,), jnp.int32),
            pltpu.VMEM((PP,), jnp.int32),
            pltpu.VMEM((PP,), jnp.int32),
            pltpu.VMEM((PP,), jnp.float32),
            pltpu.VMEM((PP,), jnp.float32),
            pltpu.VMEM((SS,), jnp.int32),
            pltpu.VMEM((SS,), jnp.int32),
            pltpu.VMEM((SS,), jnp.int32),
            pltpu.VMEM((SS,), jnp.int32),
            pltpu.VMEM((SS,), jnp.float32),
            pltpu.VMEM((SS,), jnp.float32),
            pltpu.VMEM((PP,), jnp.float32),
            pltpu.VMEM((RT,), jnp.float32),
            pltpu.SemaphoreType.DMA,
        ],
    )
    kappa_pos, kappa_neg = run(x_flat, th16, t_pos, u_pos, v_pos, us_p, vs_p)
    return kappa_pos, kappa_neg
